# baseline (speedup 1.0000x reference)
import functools

import jax
import jax.numpy as jnp
from jax import lax
from jax.experimental import pallas as pl
from jax.experimental.pallas import tpu as pltpu
from jax.experimental.pallas import tpu_sc as plsc

N, T, NP, D_IN, D, TOPK = 8, 16, 196, 384, 4, 12
NF = N * T
FB = 8
NPAD = 208
NEG = -3.0e38

NC, NS, L = 2, 16, 16
NW = NC * NS
ROWS_PER_W = NF // NW
NCHUNK = NPAD // L


NSLOT = 4


def _scores_body(x_hbm, w_ref, out_ref, xbuf, sems):
    i = pl.program_id(0)
    n_total = pl.num_programs(0)
    scale = 1.0 / jnp.sqrt(jnp.float32(D_IN))
    pad = jnp.full((NPAD - NP,), NEG, jnp.float32)
    w = w_ref[...]

    def dma(n, t, slot):
        return pltpu.make_async_copy(
            x_hbm.at[n, :, t, :], xbuf.at[slot], sems.at[slot])

    @pl.when(i == 0)
    def _():
        for s0 in range(NSLOT - 1):
            dma(0, s0, s0).start()

    for t in range(T):
        slot = t % NSLOT
        nt = t + NSLOT - 1
        if nt < T:
            dma(i, nt, nt % NSLOT).start()
        else:
            @pl.when(i + 1 < n_total)
            def _():
                dma(i + 1, nt - T, nt % NSLOT).start()
        dma(i, t, slot).wait()
        xf = xbuf[slot]
        kq = lax.dot_general(
            xf, w, (((1,), (0,)), ((), ())),
            preferred_element_type=jnp.float32,
            precision=lax.Precision.DEFAULT)
        k = kq[:, :D]
        q = kq[:, D:]
        h = lax.dot_general(
            k, q, (((1,), (1,)), ((), ())),
            preferred_element_type=jnp.float32,
            precision=lax.Precision.DEFAULT)
        s = h * scale
        m = jnp.max(s, axis=1, keepdims=True)
        e = jnp.exp(s - m)
        z = jnp.sum(e, axis=1, keepdims=True)
        a = jnp.sum(e / z, axis=0)
        out_ref[0, t, :] = jnp.concatenate([a, pad])


def _scores(xt, w):
    n = xt.shape[0]
    return pl.pallas_call(
        _scores_body,
        grid=(n,),
        in_specs=[
            pl.BlockSpec(memory_space=pltpu.MemorySpace.HBM),
            pl.BlockSpec((D_IN, 2 * D), lambda i: (0, 0)),
        ],
        out_specs=pl.BlockSpec((1, T, NPAD), lambda i: (i, 0, 0)),
        out_shape=jax.ShapeDtypeStruct((n, T, NPAD), jnp.float32),
        scratch_shapes=[
            pltpu.VMEM((NSLOT, NP, D_IN), jnp.float32),
            pltpu.SemaphoreType.DMA((NSLOT,)),
        ],
    )(xt, w)


_GDN = lax.GatherDimensionNumbers(
    offset_dims=(), collapsed_slice_dims=(0,), start_index_map=(0,))


def _shuffle(v, idx):
    return lax.gather(v, idx[:, None], _GDN, slice_sizes=(1,),
                      mode=lax.GatherScatterMode.PROMISE_IN_BOUNDS)


def _allmax(v, perms):
    for p in perms:
        v = jnp.maximum(v, _shuffle(v, p))
    return v


def _topk_body(a_hbm, out_hbm, rows_v, idx_v):
    wid = lax.axis_index("s") * NC + lax.axis_index("c")
    base = wid * ROWS_PER_W
    pltpu.sync_copy(a_hbm.at[pl.ds(base, ROWS_PER_W)], rows_v)
    lane = lax.iota(jnp.int32, L)
    perms = [lane ^ d for d in (1, 2, 4, 8)]
    for r in range(ROWS_PER_W):
        vals = [rows_v[r, pl.ds(c * L, L)] for c in range(NCHUNK)]
        idxs = [lane + c * L for c in range(NCHUNK)]
        out_vec = jnp.zeros((L,), jnp.int32)

        def step(t, carry):
            vals_c = list(carry[:NCHUNK])
            out_c = carry[NCHUNK]
            m_val, m_idx = vals_c[0], idxs[0]
            for c in range(1, NCHUNK):
                take = vals_c[c] >= m_val
                m_val = jnp.where(take, vals_c[c], m_val)
                m_idx = jnp.where(take, idxs[c], m_idx)
            gm = _allmax(m_val, perms)
            gi = _allmax(jnp.where(m_val == gm, m_idx, -1), perms)
            out_c = jnp.where(lane == t, gi, out_c)
            for c in range(NCHUNK):
                vals_c[c] = jnp.where(idxs[c] == gi, NEG, vals_c[c])
            return tuple(vals_c) + (out_c,)

        res = lax.fori_loop(0, TOPK, step, tuple(vals) + (out_vec,))
        idx_v[r, :] = res[NCHUNK]
    pltpu.sync_copy(idx_v, out_hbm.at[pl.ds(base, ROWS_PER_W)])


def _topk(a):
    mesh = plsc.VectorSubcoreMesh(core_axis_name="c", subcore_axis_name="s")
    f = functools.partial(
        pl.kernel,
        out_type=jax.ShapeDtypeStruct((NF, L), jnp.int32),
        mesh=mesh,
        scratch_types=[
            pltpu.VMEM((ROWS_PER_W, NPAD), jnp.float32),
            pltpu.VMEM((ROWS_PER_W, L), jnp.int32),
        ],
    )(_topk_body)
    return f(a)


def kernel(x, wk, wq):
    xt = jnp.transpose(x, (0, 2, 1, 3))
    w = jnp.concatenate([wk, wq], axis=1)
    a = _scores(xt, w).reshape(NF, NPAD)
    idx = _topk(a)
    return idx[:, :TOPK].reshape(N, T, TOPK, 1)


# speedup vs baseline: 2.2343x; 2.2343x over previous
import functools

import jax
import jax.numpy as jnp
from jax import lax
from jax.experimental import pallas as pl
from jax.experimental.pallas import tpu as pltpu
from jax.experimental.pallas import tpu_sc as plsc

N, T, NP, D_IN, D, TOPK = 8, 16, 196, 384, 4, 12
NF = N * T
FB = 8
NPAD = 208
NEG = -3.0e38

NC, NS, L = 2, 16, 16
NW = NC * NS
ROWS_PER_W = NF // NW
NCHUNK = NPAD // L


def _scores_body(x_ref, w_ref, out_ref):
    scale = 1.0 / jnp.sqrt(jnp.float32(D_IN))
    pad = jnp.full((NPAD - NP,), NEG, jnp.float32)
    tb = x_ref.shape[2]
    xb = x_ref[0].reshape(NP * tb, D_IN)
    kq_all = lax.dot_general(
        xb, w_ref[...], (((1,), (0,)), ((), ())),
        preferred_element_type=jnp.float32,
        precision=lax.Precision.DEFAULT).reshape(NP, tb, 2 * D)
    for t in range(tb):
        kq = kq_all[:, t, :]
        k = kq[:, :D]
        q = kq[:, D:]
        h = lax.dot_general(
            k, q, (((1,), (1,)), ((), ())),
            preferred_element_type=jnp.float32,
            precision=lax.Precision.DEFAULT)
        s = h * scale
        m = jnp.max(s, axis=1, keepdims=True)
        e = jnp.exp(s - m)
        z = jnp.sum(e, axis=1, keepdims=True)
        a = jnp.sum(e / z, axis=0)
        out_ref[0, t, :] = jnp.concatenate([a, pad])


TSPLIT = 2


def _scores(xt, w):
    n = xt.shape[0]
    tb = T // TSPLIT
    return pl.pallas_call(
        _scores_body,
        grid=(n, TSPLIT),
        in_specs=[
            pl.BlockSpec((1, NP, tb, D_IN), lambda i, j: (i, 0, j, 0)),
            pl.BlockSpec((D_IN, 2 * D), lambda i, j: (0, 0)),
        ],
        out_specs=pl.BlockSpec((1, tb, NPAD), lambda i, j: (i, j, 0)),
        out_shape=jax.ShapeDtypeStruct((n, T, NPAD), jnp.float32),
    )(xt, w)


_GDN = lax.GatherDimensionNumbers(
    offset_dims=(), collapsed_slice_dims=(0,), start_index_map=(0,))


def _shuffle(v, idx):
    return lax.gather(v, idx[:, None], _GDN, slice_sizes=(1,),
                      mode=lax.GatherScatterMode.PROMISE_IN_BOUNDS)


def _allmax(v, perms):
    for p in perms:
        v = jnp.maximum(v, _shuffle(v, p))
    return v


def _topk_body(a_hbm, out_hbm, rows_v, idx_v):
    wid = lax.axis_index("s") * NC + lax.axis_index("c")
    base = wid * ROWS_PER_W
    pltpu.sync_copy(a_hbm.at[pl.ds(base, ROWS_PER_W)], rows_v)
    lane = lax.iota(jnp.int32, L)
    perms = [lane ^ d for d in (1, 2, 4, 8)]
    for r in range(ROWS_PER_W):
        vals = [rows_v[r, pl.ds(c * L, L)] for c in range(NCHUNK)]
        idxs = [lane + c * L for c in range(NCHUNK)]
        out_vec = jnp.zeros((L,), jnp.int32)

        def step(t, carry):
            vals_c = list(carry[:NCHUNK])
            out_c = carry[NCHUNK]
            m_val, m_idx = vals_c[0], idxs[0]
            for c in range(1, NCHUNK):
                take = vals_c[c] >= m_val
                m_val = jnp.where(take, vals_c[c], m_val)
                m_idx = jnp.where(take, idxs[c], m_idx)
            gm = _allmax(m_val, perms)
            gi = _allmax(jnp.where(m_val == gm, m_idx, -1), perms)
            out_c = jnp.where(lane == t, gi, out_c)
            for c in range(NCHUNK):
                vals_c[c] = jnp.where(idxs[c] == gi, NEG, vals_c[c])
            return tuple(vals_c) + (out_c,)

        res = lax.fori_loop(0, TOPK, step, tuple(vals) + (out_vec,))
        idx_v[r, :] = res[NCHUNK]
    pltpu.sync_copy(idx_v, out_hbm.at[pl.ds(base, ROWS_PER_W)])


def _topk(a):
    mesh = plsc.VectorSubcoreMesh(core_axis_name="c", subcore_axis_name="s")
    f = functools.partial(
        pl.kernel,
        out_type=jax.ShapeDtypeStruct((NF, L), jnp.int32),
        mesh=mesh,
        scratch_types=[
            pltpu.VMEM((ROWS_PER_W, NPAD), jnp.float32),
            pltpu.VMEM((ROWS_PER_W, L), jnp.int32),
        ],
    )(_topk_body)
    return f(a)


def kernel(x, wk, wq):
    xt = jnp.transpose(x, (0, 2, 1, 3))
    w = jnp.concatenate([wk, wq], axis=1)
    a = _scores(xt, w).reshape(NF, NPAD)
    idx = _topk(a)
    return idx[:, :TOPK].reshape(N, T, TOPK, 1)


# speedup vs baseline: 2.4434x; 1.0936x over previous
import functools

import jax
import jax.numpy as jnp
from jax import lax
from jax.experimental import pallas as pl
from jax.experimental.pallas import tpu as pltpu
from jax.experimental.pallas import tpu_sc as plsc

N, T, NP, D_IN, D, TOPK = 8, 16, 196, 384, 4, 12
NF = N * T
FB = 8
NPAD = 208
NEG = -3.0e38

NC, NS, L = 2, 16, 16
NW = NC * NS
ROWS_PER_W = NF // NW
NCHUNK = NPAD // L


def _scores_body(x_ref, w_ref, out_ref):
    scale = 1.0 / jnp.sqrt(jnp.float32(D_IN))
    pad = jnp.full((NPAD - NP,), NEG, jnp.float32)
    tb = x_ref.shape[2]
    xb = x_ref[0].reshape(NP * tb, D_IN)
    kq_all = lax.dot_general(
        xb, w_ref[...], (((1,), (0,)), ((), ())),
        preferred_element_type=jnp.float32,
        precision=lax.Precision.DEFAULT).reshape(NP, tb, 2 * D)
    for t in range(tb):
        kq = kq_all[:, t, :]
        k = kq[:, :D]
        q = kq[:, D:]
        h = lax.dot_general(
            k, q, (((1,), (1,)), ((), ())),
            preferred_element_type=jnp.float32,
            precision=lax.Precision.DEFAULT)
        s = h * scale
        m = jnp.max(s, axis=1, keepdims=True)
        e = jnp.exp(s - m)
        z = jnp.sum(e, axis=1, keepdims=True)
        a = jnp.sum(e / z, axis=0)
        out_ref[0, t, :] = jnp.concatenate([a, pad])


TSPLIT = 1


def _scores(xt, w):
    n = xt.shape[0]
    tb = T // TSPLIT
    return pl.pallas_call(
        _scores_body,
        grid=(n, TSPLIT),
        in_specs=[
            pl.BlockSpec((1, NP, tb, D_IN), lambda i, j: (i, 0, j, 0)),
            pl.BlockSpec((D_IN, 2 * D), lambda i, j: (0, 0)),
        ],
        out_specs=pl.BlockSpec((1, tb, NPAD), lambda i, j: (i, j, 0)),
        out_shape=jax.ShapeDtypeStruct((n, T, NPAD), jnp.float32),
    )(xt, w)


_GDN = lax.GatherDimensionNumbers(
    offset_dims=(), collapsed_slice_dims=(0,), start_index_map=(0,))


def _shuffle(v, idx):
    return lax.gather(v, idx[:, None], _GDN, slice_sizes=(1,),
                      mode=lax.GatherScatterMode.PROMISE_IN_BOUNDS)


def _allmax(v, perms):
    for p in perms:
        v = jnp.maximum(v, _shuffle(v, p))
    return v


def _topk_body(a_hbm, out_hbm, rows_v, idx_v):
    wid = lax.axis_index("s") * NC + lax.axis_index("c")
    base = wid * ROWS_PER_W
    pltpu.sync_copy(a_hbm.at[pl.ds(base, ROWS_PER_W)], rows_v)
    lane = lax.iota(jnp.int32, L)
    perms = [lane ^ d for d in (1, 2, 4, 8)]
    for r in range(ROWS_PER_W):
        vals = [rows_v[r, pl.ds(c * L, L)] for c in range(NCHUNK)]
        idxs = [lane + c * L for c in range(NCHUNK)]
        out_vec = jnp.zeros((L,), jnp.int32)

        def step(t, carry):
            vals_c = list(carry[:NCHUNK])
            out_c = carry[NCHUNK]
            m_val, m_idx = vals_c[0], idxs[0]
            for c in range(1, NCHUNK):
                take = vals_c[c] >= m_val
                m_val = jnp.where(take, vals_c[c], m_val)
                m_idx = jnp.where(take, idxs[c], m_idx)
            gm = _allmax(m_val, perms)
            gi = _allmax(jnp.where(m_val == gm, m_idx, -1), perms)
            out_c = jnp.where(lane == t, gi, out_c)
            for c in range(NCHUNK):
                vals_c[c] = jnp.where(idxs[c] == gi, NEG, vals_c[c])
            return tuple(vals_c) + (out_c,)

        res = lax.fori_loop(0, TOPK, step, tuple(vals) + (out_vec,))
        idx_v[r, :] = res[NCHUNK]
    pltpu.sync_copy(idx_v, out_hbm.at[pl.ds(base, ROWS_PER_W)])


def _topk(a):
    mesh = plsc.VectorSubcoreMesh(core_axis_name="c", subcore_axis_name="s")
    f = functools.partial(
        pl.kernel,
        out_type=jax.ShapeDtypeStruct((NF, L), jnp.int32),
        mesh=mesh,
        scratch_types=[
            pltpu.VMEM((ROWS_PER_W, NPAD), jnp.float32),
            pltpu.VMEM((ROWS_PER_W, L), jnp.int32),
        ],
    )(_topk_body)
    return f(a)


def kernel(x, wk, wq):
    xt = jnp.transpose(x, (0, 2, 1, 3))
    w = jnp.concatenate([wk, wq], axis=1)
    a = _scores(xt, w).reshape(NF, NPAD)
    idx = _topk(a)
    return idx[:, :TOPK].reshape(N, T, TOPK, 1)


# speedup vs baseline: 2.4520x; 1.0035x over previous
import functools

import jax
import jax.numpy as jnp
from jax import lax
from jax.experimental import pallas as pl
from jax.experimental.pallas import tpu as pltpu
from jax.experimental.pallas import tpu_sc as plsc

N, T, NP, D_IN, D, TOPK = 8, 16, 196, 384, 4, 12
NF = N * T
NPAD = 208
NEG = -3.0e38

NC, NS, L = 2, 16, 16
NW = NC * NS
ROWS_PER_W = NF // NW
NCHUNK = NPAD // L


def _scores_body(x_ref, w_ref, out_ref):
    scale = 1.0 / jnp.sqrt(jnp.float32(D_IN))
    pad = jnp.full((NPAD - NP,), NEG, jnp.float32)
    tb = x_ref.shape[2]
    xb = x_ref[0].reshape(NP * tb, D_IN)
    kq_all = lax.dot_general(
        xb, w_ref[...], (((1,), (0,)), ((), ())),
        preferred_element_type=jnp.float32,
        precision=lax.Precision.DEFAULT).reshape(NP, tb, 2 * D)
    for t in range(tb):
        kq = kq_all[:, t, :]
        k = kq[:, :D]
        q = kq[:, D:]
        h = lax.dot_general(
            k, q, (((1,), (1,)), ((), ())),
            preferred_element_type=jnp.float32,
            precision=lax.Precision.DEFAULT)
        s = h * scale
        m = jnp.max(s, axis=1, keepdims=True)
        e = jnp.exp(s - m)
        z = jnp.sum(e, axis=1, keepdims=True)
        a = jnp.sum(e / z, axis=0)
        out_ref[0, t, :] = jnp.concatenate([a, pad])


TSPLIT = 1


def _scores(xt, w):
    n = xt.shape[0]
    tb = T // TSPLIT
    return pl.pallas_call(
        _scores_body,
        grid=(n, TSPLIT),
        in_specs=[
            pl.BlockSpec((1, NP, tb, D_IN), lambda i, j: (i, 0, j, 0)),
            pl.BlockSpec((D_IN, 2 * D), lambda i, j: (0, 0)),
        ],
        out_specs=pl.BlockSpec((1, tb, NPAD), lambda i, j: (i, j, 0)),
        out_shape=jax.ShapeDtypeStruct((n, T, NPAD), jnp.float32),
    )(xt, w)


_GDN = lax.GatherDimensionNumbers(
    offset_dims=(), collapsed_slice_dims=(0,), start_index_map=(0,))


def _shuffle(v, idx):
    return lax.gather(v, idx[:, None], _GDN, slice_sizes=(1,),
                      mode=lax.GatherScatterMode.PROMISE_IN_BOUNDS)


def _allmax(v, perms):
    for p in perms:
        v = jnp.maximum(v, _shuffle(v, p))
    return v


def _topk_body(a_hbm, out_hbm, rows_v, idx_v):
    wid = lax.axis_index("s") * NC + lax.axis_index("c")
    base = wid * ROWS_PER_W
    pltpu.sync_copy(a_hbm.at[pl.ds(base, ROWS_PER_W)], rows_v)
    lane = lax.iota(jnp.int32, L)
    perms = [lane ^ d for d in (1, 2, 4, 8)]
    for r in range(ROWS_PER_W):
        vals = [rows_v[r, pl.ds(c * L, L)] for c in range(NCHUNK)]
        idxs = [lane + c * L for c in range(NCHUNK)]
        out_vec = jnp.zeros((L,), jnp.int32)

        def step(t, carry):
            vals_c = list(carry[:NCHUNK])
            out_c = carry[NCHUNK]
            m_val, m_idx = vals_c[0], idxs[0]
            for c in range(1, NCHUNK):
                take = vals_c[c] >= m_val
                m_val = jnp.where(take, vals_c[c], m_val)
                m_idx = jnp.where(take, idxs[c], m_idx)
            gm = _allmax(m_val, perms)
            gi = _allmax(jnp.where(m_val == gm, m_idx, -1), perms)
            out_c = jnp.where(lane == t, gi, out_c)
            for c in range(NCHUNK):
                vals_c[c] = jnp.where(idxs[c] == gi, NEG, vals_c[c])
            return tuple(vals_c) + (out_c,)

        res = lax.fori_loop(0, TOPK, step, tuple(vals) + (out_vec,))
        idx_v[r, :] = res[NCHUNK]
    pltpu.sync_copy(idx_v, out_hbm.at[pl.ds(base, ROWS_PER_W)])


def _topk(a):
    mesh = plsc.VectorSubcoreMesh(core_axis_name="c", subcore_axis_name="s")
    f = functools.partial(
        pl.kernel,
        out_type=jax.ShapeDtypeStruct((NF, L), jnp.int32),
        mesh=mesh,
        scratch_types=[
            pltpu.VMEM((ROWS_PER_W, NPAD), jnp.float32),
            pltpu.VMEM((ROWS_PER_W, L), jnp.int32),
        ],
    )(_topk_body)
    return f(a)


def kernel(x, wk, wq):
    xt = jnp.transpose(x, (0, 2, 1, 3))
    w = jnp.concatenate([wk, wq], axis=1)
    a = _scores(xt, w).reshape(NF, NPAD)
    idx = _topk(a)
    return idx[:, :TOPK].reshape(N, T, TOPK, 1)
